# Initial kernel scaffold; baseline (speedup 1.0000x reference)
import jax
import jax.numpy as jnp
from jax.experimental import pallas as pl


def kernel(V, E, edge_index, U, Vm, P, Pb, W1, b1, W2, b2):
    raise NotImplementedError("write your pallas kernel here")


# speedup vs baseline: 2.6713x; 2.6713x over previous
import functools

import jax
import jax.numpy as jnp
from jax import lax
from jax.experimental import pallas as pl
from jax.experimental.pallas import tpu as pltpu
from jax.experimental.pallas import tpu_sc as plsc

_NC = 2
_NS = 16
_NW = _NC * _NS


def _pre_body(v_ref, ut_ref, vmt_ref, a_ref, b_ref):
    v = v_ref[...]
    a_ref[...] = jnp.dot(v, ut_ref[...], preferred_element_type=jnp.float32)
    b_ref[...] = jnp.dot(v, vmt_ref[...], preferred_element_type=jnp.float32)


def _precompute(V, Ut, Vmt):
    n, d = V.shape
    k = Ut.shape[1]
    return pl.pallas_call(
        _pre_body,
        out_shape=(
            jax.ShapeDtypeStruct((n, k), jnp.float32),
            jax.ShapeDtypeStruct((n, k), jnp.float32),
        ),
    )(V, Ut, Vmt)


def _gather_body(chunk, n_iter, a_hbm, b_hbm, src_hbm, dst_hbm,
                 oa_hbm, ob_hbm, idx_s, idx_d, rows_a, rows_b,
                 sem_a, sem_b):
    wid = lax.axis_index("s") * _NC + lax.axis_index("c")
    per_w = chunk * n_iter
    for it in range(n_iter):
        base = pl.multiple_of(wid * per_w + it * chunk, 8)
        pltpu.sync_copy(src_hbm.at[pl.ds(base, chunk)], idx_s)
        pltpu.sync_copy(dst_hbm.at[pl.ds(base, chunk)], idx_d)
        cp_a = pltpu.async_copy(a_hbm.at[idx_s], rows_a, sem_a)
        cp_b = pltpu.async_copy(b_hbm.at[idx_d], rows_b, sem_b)
        cp_a.wait()
        cp_b.wait()
        pltpu.sync_copy(rows_a, oa_hbm.at[pl.ds(base, chunk)])
        pltpu.sync_copy(rows_b, ob_hbm.at[pl.ds(base, chunk)])


def _sc_gather(A, B, src, dst):
    n_edges = src.shape[0]
    k = A.shape[1]
    per_w = n_edges // _NW
    chunk = 2000
    n_iter = per_w // chunk
    assert per_w * _NW == n_edges and chunk * n_iter == per_w

    mesh = plsc.VectorSubcoreMesh(
        core_axis_name="c", subcore_axis_name="s",
        num_cores=_NC, num_subcores=_NS)
    body = functools.partial(_gather_body, chunk, n_iter)
    return pl.kernel(
        body,
        out_type=(
            jax.ShapeDtypeStruct((n_edges, k), jnp.float32),
            jax.ShapeDtypeStruct((n_edges, k), jnp.float32),
        ),
        mesh=mesh,
        scratch_types=[
            pltpu.VMEM((chunk,), jnp.int32),
            pltpu.VMEM((chunk,), jnp.int32),
            pltpu.VMEM((chunk, k), jnp.float32),
            pltpu.VMEM((chunk, k), jnp.float32),
            pltpu.SemaphoreType.DMA,
            pltpu.SemaphoreType.DMA,
        ],
        compiler_params=pltpu.CompilerParams(use_tc_tiling_on_sc=False),
    )(A, B, src, dst)


def _mlp_body(a_ref, b_ref, e_ref, q_ref, w1e_ref, w2t_ref, c1_ref,
              b2_ref, out_ref):
    t = jnp.tanh(a_ref[...] * b_ref[...])
    e = e_ref[...]
    ea = jnp.where(e >= 0.0, e, 0.01 * e)
    x = (jnp.dot(t, q_ref[...], preferred_element_type=jnp.float32)
         + jnp.dot(ea, w1e_ref[...], preferred_element_type=jnp.float32)
         + c1_ref[...])
    y = jnp.maximum(x, 0.0)
    out_ref[...] = (jnp.dot(y, w2t_ref[...],
                            preferred_element_type=jnp.float32)
                    + b2_ref[...])


def _mlp(Asrc, Bdst, E, Q, W1eT, W2T, c1, b2r):
    n_edges, k = Asrc.shape
    blk = 4000
    grid = n_edges // blk
    assert blk * grid == n_edges
    edge_spec = pl.BlockSpec((blk, k), lambda i: (i, 0))
    w_spec = pl.BlockSpec((k, k), lambda i: (0, 0))
    bias_spec = pl.BlockSpec((1, k), lambda i: (0, 0))
    return pl.pallas_call(
        _mlp_body,
        grid=(grid,),
        in_specs=[edge_spec, edge_spec, edge_spec,
                  w_spec, w_spec, w_spec, bias_spec, bias_spec],
        out_specs=edge_spec,
        out_shape=jax.ShapeDtypeStruct((n_edges, k), jnp.float32),
    )(Asrc, Bdst, E, Q, W1eT, W2T, c1, b2r)


def kernel(V, E, edge_index, U, Vm, P, Pb, W1, b1, W2, b2):
    src = edge_index[0]
    dst = edge_index[1]
    A, B = _precompute(V, U.T, Vm.T)
    Asrc, Bdst = _sc_gather(A, B, src, dst)
    k = P.shape[0]
    W1h_T = W1[:, :k].T
    Q = P.T @ W1h_T
    c1 = (Pb @ W1h_T + b1).reshape(1, k)
    W1eT = W1[:, k:].T
    return _mlp(Asrc, Bdst, E, Q, W1eT, W2.T, c1, b2.reshape(1, k))


# speedup vs baseline: 4.8098x; 1.8005x over previous
import functools

import jax
import jax.numpy as jnp
from jax import lax
from jax.experimental import pallas as pl
from jax.experimental.pallas import tpu as pltpu
from jax.experimental.pallas import tpu_sc as plsc

_NC = 2
_NS = 16
_NW = _NC * _NS


def _pre_body(v_ref, ut_ref, vmt_ref, a_ref, b_ref):
    v = v_ref[...]
    a_ref[...] = jnp.dot(v, ut_ref[...], preferred_element_type=jnp.float32)
    b_ref[...] = jnp.dot(v, vmt_ref[...], preferred_element_type=jnp.float32)


def _precompute(V, Ut, Vmt):
    n, d = V.shape
    k = Ut.shape[1]
    return pl.pallas_call(
        _pre_body,
        out_shape=(
            jax.ShapeDtypeStruct((n, k), jnp.float32),
            jax.ShapeDtypeStruct((n, k), jnp.float32),
        ),
    )(V, Ut, Vmt)


def _gather_body(chunk, n_iter, a_hbm, b_hbm, src_hbm, dst_hbm,
                 oa_hbm, ob_hbm, idx_s, idx_d, rows_a, rows_b,
                 sem_a, sem_b):
    wid = lax.axis_index("s") * _NC + lax.axis_index("c")
    per_w = chunk * n_iter
    for it in range(n_iter):
        base = pl.multiple_of(wid * per_w + it * chunk, 8)
        pltpu.sync_copy(src_hbm.at[pl.ds(base, chunk)], idx_s)
        pltpu.sync_copy(dst_hbm.at[pl.ds(base, chunk)], idx_d)
        cp_a = pltpu.async_copy(a_hbm.at[idx_s], rows_a, sem_a)
        cp_b = pltpu.async_copy(b_hbm.at[idx_d], rows_b, sem_b)
        cp_a.wait()
        cp_b.wait()
        pltpu.sync_copy(rows_a, oa_hbm.at[pl.ds(base, chunk)])
        pltpu.sync_copy(rows_b, ob_hbm.at[pl.ds(base, chunk)])


def _sc_gather(A, B, src, dst):
    n_edges = src.shape[0]
    k = A.shape[1]
    per_w = n_edges // _NW
    chunk = 2000
    n_iter = per_w // chunk
    assert per_w * _NW == n_edges and chunk * n_iter == per_w

    mesh = plsc.VectorSubcoreMesh(
        core_axis_name="c", subcore_axis_name="s",
        num_cores=_NC, num_subcores=_NS)
    body = functools.partial(_gather_body, chunk, n_iter)
    return pl.kernel(
        body,
        out_type=(
            jax.ShapeDtypeStruct((n_edges, k), jnp.float32),
            jax.ShapeDtypeStruct((n_edges, k), jnp.float32),
        ),
        mesh=mesh,
        scratch_types=[
            pltpu.VMEM((chunk,), jnp.int32),
            pltpu.VMEM((chunk,), jnp.int32),
            pltpu.VMEM((chunk, k), jnp.float32),
            pltpu.VMEM((chunk, k), jnp.float32),
            pltpu.SemaphoreType.DMA,
            pltpu.SemaphoreType.DMA,
        ],
        compiler_params=pltpu.CompilerParams(use_tc_tiling_on_sc=False),
    )(A, B, src, dst)


def _mlp_body(a_ref, b_ref, e_ref, q_ref, w1e_ref, w2t_ref, c1_ref,
              b2_ref, out_ref):
    t = jnp.tanh(a_ref[...] * b_ref[...])
    e = e_ref[...]
    ea = jnp.where(e >= 0.0, e, 0.01 * e)
    x = (jnp.dot(t, q_ref[...], preferred_element_type=jnp.float32)
         + jnp.dot(ea, w1e_ref[...], preferred_element_type=jnp.float32)
         + c1_ref[...])
    y = jnp.maximum(x, 0.0)
    out_ref[...] = (jnp.dot(y, w2t_ref[...],
                            preferred_element_type=jnp.float32)
                    + b2_ref[...])


def _mlp(Asrc2, Bdst2, E2, Q8, W1eT8, W2T8, c1_8, b2_8):
    n_rows, width = Asrc2.shape
    blk = 4000
    grid = n_rows // blk
    assert blk * grid == n_rows
    edge_spec = pl.BlockSpec((blk, width), lambda i: (i, 0))
    w_spec = pl.BlockSpec((width, width), lambda i: (0, 0))
    bias_spec = pl.BlockSpec((1, width), lambda i: (0, 0))
    return pl.pallas_call(
        _mlp_body,
        grid=(grid,),
        in_specs=[edge_spec, edge_spec, edge_spec,
                  w_spec, w_spec, w_spec, bias_spec, bias_spec],
        out_specs=edge_spec,
        out_shape=jax.ShapeDtypeStruct((n_rows, width), jnp.float32),
    )(Asrc2, Bdst2, E2, Q8, W1eT8, W2T8, c1_8, b2_8)


def kernel(V, E, edge_index, U, Vm, P, Pb, W1, b1, W2, b2):
    src = edge_index[0]
    dst = edge_index[1]
    A, B = _precompute(V, U.T, Vm.T)
    Asrc, Bdst = _sc_gather(A, B, src, dst)
    k = P.shape[0]
    W1h_T = W1[:, :k].T
    Q = P.T @ W1h_T
    c1 = Pb @ W1h_T + b1
    W1eT = W1[:, k:].T
    n_edges = E.shape[0]
    pack = 128 // k
    n_rows = n_edges // pack
    eye = jnp.eye(pack, dtype=jnp.float32)
    out2 = _mlp(
        Asrc.reshape(n_rows, 128), Bdst.reshape(n_rows, 128),
        E.reshape(n_rows, 128),
        jnp.kron(eye, Q), jnp.kron(eye, W1eT), jnp.kron(eye, W2.T),
        jnp.tile(c1, pack).reshape(1, 128),
        jnp.tile(b2, pack).reshape(1, 128))
    return out2.reshape(n_edges, k)


# speedup vs baseline: 5.9824x; 1.2438x over previous
import jax
import jax.numpy as jnp
from jax import lax
from jax.experimental import pallas as pl
from jax.experimental.pallas import tpu as pltpu
from jax.experimental.pallas import tpu_sc as plsc

_NC = 2
_NS = 16
_NW = _NC * _NS
_K = 16
_CHUNK = 2000


def _pre_body(v_ref, ut_ref, vmt_ref, a_ref, b_ref):
    v = v_ref[...]
    a_ref[...] = jnp.dot(v, ut_ref[...], preferred_element_type=jnp.float32)
    b_ref[...] = jnp.dot(v, vmt_ref[...], preferred_element_type=jnp.float32)


def _precompute(V, Ut, Vmt):
    n, _ = V.shape
    k = Ut.shape[1]
    return pl.pallas_call(
        _pre_body,
        out_shape=(
            jax.ShapeDtypeStruct((n, k), jnp.float32),
            jax.ShapeDtypeStruct((n, k), jnp.float32),
        ),
    )(V, Ut, Vmt)


def _gather_body(a_hbm, b_hbm, src_hbm, dst_hbm, oa_hbm, ob_hbm,
                 idx_s, idx_d, rows_a, rows_b, sem_a, sem_b):
    wid = lax.axis_index("s") * _NC + lax.axis_index("c")
    n_edges = src_hbm.shape[0]
    per_w = n_edges // _NW
    n_iter = per_w // _CHUNK

    def chunk_fn(jj, carry):
        base = pl.multiple_of(wid * per_w + jj * _CHUNK, 8)
        pltpu.sync_copy(src_hbm.at[pl.ds(base, _CHUNK)], idx_s)
        pltpu.sync_copy(dst_hbm.at[pl.ds(base, _CHUNK)], idx_d)
        cp_a = pltpu.async_copy(a_hbm.at[idx_s], rows_a, sem_a)
        cp_b = pltpu.async_copy(b_hbm.at[idx_d], rows_b, sem_b)
        cp_a.wait()
        cp_b.wait()
        pltpu.sync_copy(rows_a, oa_hbm.at[pl.ds(base, _CHUNK)])
        pltpu.sync_copy(rows_b, ob_hbm.at[pl.ds(base, _CHUNK)])
        return carry

    lax.fori_loop(0, n_iter, chunk_fn, 0)


def _sc_gather(A, B, src, dst):
    n_edges = src.shape[0]
    k = A.shape[1]
    assert (n_edges // _NW) % _CHUNK == 0
    mesh = plsc.VectorSubcoreMesh(
        core_axis_name="c", subcore_axis_name="s",
        num_cores=_NC, num_subcores=_NS)
    return pl.kernel(
        _gather_body,
        out_type=(
            jax.ShapeDtypeStruct((n_edges, k), jnp.float32),
            jax.ShapeDtypeStruct((n_edges, k), jnp.float32),
        ),
        mesh=mesh,
        scratch_types=[
            pltpu.VMEM((_CHUNK,), jnp.int32),
            pltpu.VMEM((_CHUNK,), jnp.int32),
            pltpu.VMEM((_CHUNK, _K), jnp.float32),
            pltpu.VMEM((_CHUNK, _K), jnp.float32),
            pltpu.SemaphoreType.DMA,
            pltpu.SemaphoreType.DMA,
        ],
        compiler_params=pltpu.CompilerParams(use_tc_tiling_on_sc=False),
    )(A, B, src, dst)


def _tanh_body(a_ref, b_ref, t_ref):
    t_ref[...] = jnp.tanh(a_ref[...] * b_ref[...])


def _tanh_packed(A2, B2):
    n_rows, width = A2.shape
    blk = 4000
    grid = n_rows // blk
    spec = pl.BlockSpec((blk, width), lambda i: (i, 0))
    return pl.pallas_call(
        _tanh_body,
        grid=(grid,),
        in_specs=[spec, spec],
        out_specs=spec,
        out_shape=jax.ShapeDtypeStruct((n_rows, width), jnp.float32),
    )(A2, B2)


def _mlp_body(t_ref, e_ref, qt_ref, w1e_ref, w2_ref, c1_ref, b2_ref,
              out_ref):
    e = e_ref[...]
    ea = jnp.where(e >= 0.0, e, 0.01 * e)
    x = (jnp.dot(qt_ref[...], t_ref[...], preferred_element_type=jnp.float32)
         + jnp.dot(w1e_ref[...], ea, preferred_element_type=jnp.float32)
         + c1_ref[...])
    y = jnp.maximum(x, 0.0)
    out_ref[...] = (jnp.dot(w2_ref[...], y,
                            preferred_element_type=jnp.float32)
                    + b2_ref[...])


def _mlp_t(T16, ET, QT, W1E, W2, c1c, b2c):
    k, n_edges = T16.shape
    nb = 32000
    grid = n_edges // nb
    edge_spec = pl.BlockSpec((k, nb), lambda i: (0, i))
    w_spec = pl.BlockSpec((k, k), lambda i: (0, 0))
    bias_spec = pl.BlockSpec((k, 1), lambda i: (0, 0))
    return pl.pallas_call(
        _mlp_body,
        grid=(grid,),
        in_specs=[edge_spec, edge_spec,
                  w_spec, w_spec, w_spec, bias_spec, bias_spec],
        out_specs=edge_spec,
        out_shape=jax.ShapeDtypeStruct((k, n_edges), jnp.float32),
    )(T16, ET, QT, W1E, W2, c1c, b2c)


def kernel(V, E, edge_index, U, Vm, P, Pb, W1, b1, W2, b2):
    src = edge_index[0]
    dst = edge_index[1]
    n_edges, k = E.shape
    A, B = _precompute(V, U.T, Vm.T)
    Asrc, Bdst = _sc_gather(A, B, src, dst)
    pack = 128 // k
    n_rows = n_edges // pack
    t_p = _tanh_packed(Asrc.reshape(n_rows, 128), Bdst.reshape(n_rows, 128))
    T16 = t_p.reshape(n_edges, k).T
    ET = E.T
    W1h_T = W1[:, :k].T
    Q = P.T @ W1h_T
    c1 = Pb @ W1h_T + b1
    out_t = _mlp_t(T16, ET, Q.T, W1[:, k:], W2,
                   c1.reshape(k, 1), b2.reshape(k, 1))
    return out_t.T


# speedup vs baseline: 6.3598x; 1.0631x over previous
import jax
import jax.numpy as jnp
from jax import lax
from jax.experimental import pallas as pl
from jax.experimental.pallas import tpu as pltpu
from jax.experimental.pallas import tpu_sc as plsc

_NC = 2
_NS = 16
_NW = _NC * _NS
_K = 16
_CHUNK = 2000


def _pre_body(v_ref, ut_ref, vmt_ref, a_ref, b_ref):
    v = v_ref[...]
    a_ref[...] = jnp.dot(v, ut_ref[...], preferred_element_type=jnp.float32)
    b_ref[...] = jnp.dot(v, vmt_ref[...], preferred_element_type=jnp.float32)


def _precompute(V, Ut, Vmt):
    n, _ = V.shape
    k = Ut.shape[1]
    return pl.pallas_call(
        _pre_body,
        out_shape=(
            jax.ShapeDtypeStruct((n, k), jnp.float32),
            jax.ShapeDtypeStruct((n, k), jnp.float32),
        ),
    )(V, Ut, Vmt)


def _gather_body(a_hbm, b_hbm, src_hbm, dst_hbm, oa_hbm, ob_hbm,
                 idx_s, idx_d, rows_a, rows_b, sem_a, sem_b):
    wid = lax.axis_index("s") * _NC + lax.axis_index("c")
    n_edges = src_hbm.shape[0]
    per_w = n_edges // _NW
    n_iter = per_w // _CHUNK

    def chunk_fn(jj, carry):
        base = pl.multiple_of(wid * per_w + jj * _CHUNK, 8)
        pltpu.sync_copy(src_hbm.at[pl.ds(base, _CHUNK)], idx_s)
        pltpu.sync_copy(dst_hbm.at[pl.ds(base, _CHUNK)], idx_d)
        cp_a = pltpu.async_copy(a_hbm.at[idx_s], rows_a, sem_a)
        cp_b = pltpu.async_copy(b_hbm.at[idx_d], rows_b, sem_b)
        cp_a.wait()
        cp_b.wait()
        pltpu.sync_copy(rows_a, oa_hbm.at[pl.ds(base, _CHUNK)])
        pltpu.sync_copy(rows_b, ob_hbm.at[pl.ds(base, _CHUNK)])
        return carry

    lax.fori_loop(0, n_iter, chunk_fn, 0)


def _sc_gather(A, B, src, dst):
    n_edges = src.shape[0]
    k = A.shape[1]
    assert (n_edges // _NW) % _CHUNK == 0
    mesh = plsc.VectorSubcoreMesh(
        core_axis_name="c", subcore_axis_name="s",
        num_cores=_NC, num_subcores=_NS)
    return pl.kernel(
        _gather_body,
        out_type=(
            jax.ShapeDtypeStruct((n_edges, k), jnp.float32),
            jax.ShapeDtypeStruct((n_edges, k), jnp.float32),
        ),
        mesh=mesh,
        scratch_types=[
            pltpu.VMEM((_CHUNK,), jnp.int32),
            pltpu.VMEM((_CHUNK,), jnp.int32),
            pltpu.VMEM((_CHUNK, _K), jnp.float32),
            pltpu.VMEM((_CHUNK, _K), jnp.float32),
            pltpu.SemaphoreType.DMA,
            pltpu.SemaphoreType.DMA,
        ],
        compiler_params=pltpu.CompilerParams(use_tc_tiling_on_sc=False),
    )(A, B, src, dst)


def _tanh_body(a_ref, b_ref, t_ref):
    t_ref[...] = jnp.tanh(a_ref[...] * b_ref[...]).astype(jnp.bfloat16)


def _tanh_packed(A2, B2):
    n_rows, width = A2.shape
    blk = 4000
    grid = n_rows // blk
    spec = pl.BlockSpec((blk, width), lambda i: (i, 0))
    return pl.pallas_call(
        _tanh_body,
        grid=(grid,),
        in_specs=[spec, spec],
        out_specs=spec,
        out_shape=jax.ShapeDtypeStruct((n_rows, width), jnp.bfloat16),
    )(A2, B2)


def _mlp_body(t_ref, e_ref, qt_ref, w1e_ref, w2_ref, c1_ref, b2_ref,
              out_ref):
    e = e_ref[...]
    ea = jnp.where(e >= 0.0, e, 0.01 * e)
    t = t_ref[...].astype(jnp.float32)
    x = (jnp.dot(qt_ref[...], t, preferred_element_type=jnp.float32)
         + jnp.dot(w1e_ref[...], ea, preferred_element_type=jnp.float32)
         + c1_ref[...])
    y = jnp.maximum(x, 0.0)
    out_ref[...] = (jnp.dot(w2_ref[...], y,
                            preferred_element_type=jnp.float32)
                    + b2_ref[...])


def _mlp_t(T16, ET, QT, W1E, W2, c1c, b2c):
    k, n_edges = T16.shape
    nb = 32000
    grid = n_edges // nb
    edge_spec = pl.BlockSpec((k, nb), lambda i: (0, i))
    w_spec = pl.BlockSpec((k, k), lambda i: (0, 0))
    bias_spec = pl.BlockSpec((k, 1), lambda i: (0, 0))
    return pl.pallas_call(
        _mlp_body,
        grid=(grid,),
        in_specs=[edge_spec, edge_spec,
                  w_spec, w_spec, w_spec, bias_spec, bias_spec],
        out_specs=edge_spec,
        out_shape=jax.ShapeDtypeStruct((k, n_edges), jnp.float32),
    )(T16, ET, QT, W1E, W2, c1c, b2c)


def kernel(V, E, edge_index, U, Vm, P, Pb, W1, b1, W2, b2):
    src = edge_index[0]
    dst = edge_index[1]
    n_edges, k = E.shape
    A, B = _precompute(V, U.T, Vm.T)
    Asrc, Bdst = _sc_gather(A, B, src, dst)
    pack = 128 // k
    n_rows = n_edges // pack
    t_p = _tanh_packed(Asrc.reshape(n_rows, 128), Bdst.reshape(n_rows, 128))
    T16 = t_p.reshape(n_edges, k).T
    ET = E.T
    W1h_T = W1[:, :k].T
    Q = P.T @ W1h_T
    c1 = Pb @ W1h_T + b1
    out_t = _mlp_t(T16, ET, Q.T, W1[:, k:], W2,
                   c1.reshape(k, 1), b2.reshape(k, 1))
    return out_t.T


# speedup vs baseline: 6.4010x; 1.0065x over previous
import jax
import jax.numpy as jnp
from jax import lax
from jax.experimental import pallas as pl
from jax.experimental.pallas import tpu as pltpu
from jax.experimental.pallas import tpu_sc as plsc

_NC = 2
_NS = 16
_NW = _NC * _NS
_K = 16
_CHUNK = 1000


def _pre_body(v_ref, ut_ref, vmt_ref, a_ref, b_ref):
    v = v_ref[...]
    a_ref[...] = jnp.dot(v, ut_ref[...], preferred_element_type=jnp.float32)
    b_ref[...] = jnp.dot(v, vmt_ref[...], preferred_element_type=jnp.float32)


def _precompute(V, Ut, Vmt):
    n, _ = V.shape
    k = Ut.shape[1]
    return pl.pallas_call(
        _pre_body,
        out_shape=(
            jax.ShapeDtypeStruct((n, k), jnp.float32),
            jax.ShapeDtypeStruct((n, k), jnp.float32),
        ),
    )(V, Ut, Vmt)


def _gather_body(a_hbm, b_hbm, src_hbm, dst_hbm, oa_hbm, ob_hbm,
                 idx_s, idx_d, rows_a, rows_b, sem_a, sem_b):
    wid = lax.axis_index("s") * _NC + lax.axis_index("c")
    n_edges = src_hbm.shape[0]
    per_w = n_edges // _NW
    n_iter = per_w // _CHUNK

    def chunk_fn(jj, carry):
        base = pl.multiple_of(wid * per_w + jj * _CHUNK, 8)
        pltpu.sync_copy(src_hbm.at[pl.ds(base, _CHUNK)], idx_s)
        pltpu.sync_copy(dst_hbm.at[pl.ds(base, _CHUNK)], idx_d)
        cp_a = pltpu.async_copy(a_hbm.at[idx_s], rows_a, sem_a)
        cp_b = pltpu.async_copy(b_hbm.at[idx_d], rows_b, sem_b)
        cp_a.wait()
        cp_b.wait()
        pltpu.sync_copy(rows_a, oa_hbm.at[pl.ds(base, _CHUNK)])
        pltpu.sync_copy(rows_b, ob_hbm.at[pl.ds(base, _CHUNK)])
        return carry

    lax.fori_loop(0, n_iter, chunk_fn, 0)


def _sc_gather(A, B, src, dst):
    n_edges = src.shape[0]
    k = A.shape[1]
    assert (n_edges // _NW) % _CHUNK == 0
    mesh = plsc.VectorSubcoreMesh(
        core_axis_name="c", subcore_axis_name="s",
        num_cores=_NC, num_subcores=_NS)
    return pl.kernel(
        _gather_body,
        out_type=(
            jax.ShapeDtypeStruct((n_edges, k), jnp.float32),
            jax.ShapeDtypeStruct((n_edges, k), jnp.float32),
        ),
        mesh=mesh,
        scratch_types=[
            pltpu.VMEM((_CHUNK,), jnp.int32),
            pltpu.VMEM((_CHUNK,), jnp.int32),
            pltpu.VMEM((_CHUNK, _K), jnp.float32),
            pltpu.VMEM((_CHUNK, _K), jnp.float32),
            pltpu.SemaphoreType.DMA,
            pltpu.SemaphoreType.DMA,
        ],
        compiler_params=pltpu.CompilerParams(use_tc_tiling_on_sc=False),
    )(A, B, src, dst)


def _tanh_body(a_ref, b_ref, t_ref):
    t_ref[...] = jnp.tanh(a_ref[...] * b_ref[...]).astype(jnp.bfloat16)


def _tanh_packed(A2, B2):
    n_rows, width = A2.shape
    blk = 4000
    grid = n_rows // blk
    spec = pl.BlockSpec((blk, width), lambda i: (i, 0))
    return pl.pallas_call(
        _tanh_body,
        grid=(grid,),
        in_specs=[spec, spec],
        out_specs=spec,
        out_shape=jax.ShapeDtypeStruct((n_rows, width), jnp.bfloat16),
    )(A2, B2)


def _mlp_body(nb0, t0_ref, t1_ref, e_ref, qt_ref, w1e_ref, w2_ref,
              c1_ref, b2_ref, out_ref):
    i = pl.program_id(0)
    t_bf = jnp.where(i < nb0, t0_ref[...], t1_ref[...])
    t = t_bf.astype(jnp.float32)
    e = e_ref[...]
    ea = jnp.where(e >= 0.0, e, 0.01 * e)
    x = (jnp.dot(qt_ref[...], t, preferred_element_type=jnp.float32)
         + jnp.dot(w1e_ref[...], ea, preferred_element_type=jnp.float32)
         + c1_ref[...])
    y = jnp.maximum(x, 0.0)
    out_ref[...] = (jnp.dot(w2_ref[...], y,
                            preferred_element_type=jnp.float32)
                    + b2_ref[...])


def _mlp_t(T0, T1, ET, QT, W1E, W2, c1c, b2c):
    import functools
    k, n_edges = ET.shape
    nb = 16000
    grid = n_edges // nb
    nb0 = T0.shape[1] // nb
    t0_spec = pl.BlockSpec((k, nb), lambda i: (0, jnp.minimum(i, nb0 - 1)))
    t1_spec = pl.BlockSpec(
        (k, nb), lambda i: (0, jnp.maximum(i - nb0, 0)))
    edge_spec = pl.BlockSpec((k, nb), lambda i: (0, i))
    w_spec = pl.BlockSpec((k, k), lambda i: (0, 0))
    bias_spec = pl.BlockSpec((k, 1), lambda i: (0, 0))
    return pl.pallas_call(
        functools.partial(_mlp_body, nb0),
        grid=(grid,),
        in_specs=[t0_spec, t1_spec, edge_spec,
                  w_spec, w_spec, w_spec, bias_spec, bias_spec],
        out_specs=edge_spec,
        out_shape=jax.ShapeDtypeStruct((k, n_edges), jnp.float32),
    )(T0, T1, ET, QT, W1E, W2, c1c, b2c)


def kernel(V, E, edge_index, U, Vm, P, Pb, W1, b1, W2, b2):
    n_edges, k = E.shape
    half = n_edges // 2
    pack = 128 // k
    A, B = _precompute(V, U.T, Vm.T)

    t16s = []
    for h in range(2):
        src = lax.slice(edge_index, (0, h * half), (1, (h + 1) * half))
        dst = lax.slice(edge_index, (1, h * half), (2, (h + 1) * half))
        Asrc, Bdst = _sc_gather(A, B, src.reshape(half), dst.reshape(half))
        t_p = _tanh_packed(Asrc.reshape(half // pack, 128),
                           Bdst.reshape(half // pack, 128))
        t16s.append(t_p.reshape(half, k).T)

    ET = E.T
    W1h_T = W1[:, :k].T
    Q = P.T @ W1h_T
    c1 = Pb @ W1h_T + b1
    out_t = _mlp_t(t16s[0], t16s[1], ET, Q.T, W1[:, k:], W2,
                   c1.reshape(k, 1), b2.reshape(k, 1))
    return out_t.T


# speedup vs baseline: 6.5709x; 1.0265x over previous
import jax
import jax.numpy as jnp
from jax import lax
from jax.experimental import pallas as pl
from jax.experimental.pallas import tpu as pltpu
from jax.experimental.pallas import tpu_sc as plsc

_NC = 2
_NS = 16
_NW = _NC * _NS
_K = 16
_CHUNK = 1000


def _pre_body(v_ref, ut_ref, vmt_ref, a_ref, b_ref):
    v = v_ref[...]
    a_ref[...] = jnp.dot(v, ut_ref[...], preferred_element_type=jnp.float32)
    b_ref[...] = jnp.dot(v, vmt_ref[...], preferred_element_type=jnp.float32)


def _precompute(V, Ut, Vmt):
    n, _ = V.shape
    k = Ut.shape[1]
    return pl.pallas_call(
        _pre_body,
        out_shape=(
            jax.ShapeDtypeStruct((n, k), jnp.float32),
            jax.ShapeDtypeStruct((n, k), jnp.float32),
        ),
    )(V, Ut, Vmt)


def _gather_body(h_base, n_edges, a_hbm, b_hbm, ei_hbm, oa_hbm, ob_hbm,
                 idx_s, idx_d, rows_a, rows_b, sem_a, sem_b):
    wid = lax.axis_index("s") * _NC + lax.axis_index("c")
    per_w = n_edges // _NW
    n_iter = per_w // _CHUNK

    def chunk_fn(jj, carry):
        base = pl.multiple_of(wid * per_w + jj * _CHUNK, 8)
        pltpu.sync_copy(ei_hbm.at[0, pl.ds(h_base + base, _CHUNK)], idx_s)
        pltpu.sync_copy(ei_hbm.at[1, pl.ds(h_base + base, _CHUNK)], idx_d)
        cp_a = pltpu.async_copy(a_hbm.at[idx_s], rows_a, sem_a)
        cp_b = pltpu.async_copy(b_hbm.at[idx_d], rows_b, sem_b)
        cp_a.wait()
        cp_b.wait()
        pltpu.sync_copy(rows_a, oa_hbm.at[pl.ds(base, _CHUNK)])
        pltpu.sync_copy(rows_b, ob_hbm.at[pl.ds(base, _CHUNK)])
        return carry

    lax.fori_loop(0, n_iter, chunk_fn, 0)


def _sc_gather(A, B, ei, h_base, n_edges):
    import functools
    k = A.shape[1]
    assert (n_edges // _NW) % _CHUNK == 0
    mesh = plsc.VectorSubcoreMesh(
        core_axis_name="c", subcore_axis_name="s",
        num_cores=_NC, num_subcores=_NS)
    return pl.kernel(
        functools.partial(_gather_body, h_base, n_edges),
        out_type=(
            jax.ShapeDtypeStruct((n_edges, k), jnp.float32),
            jax.ShapeDtypeStruct((n_edges, k), jnp.float32),
        ),
        mesh=mesh,
        scratch_types=[
            pltpu.VMEM((_CHUNK,), jnp.int32),
            pltpu.VMEM((_CHUNK,), jnp.int32),
            pltpu.VMEM((_CHUNK, _K), jnp.float32),
            pltpu.VMEM((_CHUNK, _K), jnp.float32),
            pltpu.SemaphoreType.DMA,
            pltpu.SemaphoreType.DMA,
        ],
        compiler_params=pltpu.CompilerParams(use_tc_tiling_on_sc=False),
    )(A, B, ei)


def _tanh_body(a_ref, b_ref, t_ref):
    t_ref[...] = jnp.tanh(a_ref[...] * b_ref[...]).astype(jnp.bfloat16)


def _tanh_packed(A2, B2):
    n_rows, width = A2.shape
    blk = 4000
    grid = n_rows // blk
    spec = pl.BlockSpec((blk, width), lambda i: (i, 0))
    return pl.pallas_call(
        _tanh_body,
        grid=(grid,),
        in_specs=[spec, spec],
        out_specs=spec,
        out_shape=jax.ShapeDtypeStruct((n_rows, width), jnp.bfloat16),
    )(A2, B2)


def _mlp_body(nb0, t0_ref, t1_ref, e_ref, qt_ref, w1e_ref, w2_ref,
              c1_ref, b2_ref, out_ref):
    i = pl.program_id(0)
    t_bf = jnp.where(i < nb0, t0_ref[...], t1_ref[...])
    t = t_bf.astype(jnp.float32)
    e = e_ref[...]
    ea = jnp.where(e >= 0.0, e, 0.01 * e)
    x = (jnp.dot(qt_ref[...], t, preferred_element_type=jnp.float32)
         + jnp.dot(w1e_ref[...], ea, preferred_element_type=jnp.float32)
         + c1_ref[...])
    y = jnp.maximum(x, 0.0)
    out_ref[...] = (jnp.dot(w2_ref[...], y,
                            preferred_element_type=jnp.float32)
                    + b2_ref[...])


def _mlp_t(T0, T1, ET, QT, W1E, W2, c1c, b2c):
    import functools
    k, n_edges = ET.shape
    nb = 32000
    grid = n_edges // nb
    nb0 = T0.shape[1] // nb
    t0_spec = pl.BlockSpec((k, nb), lambda i: (0, jnp.minimum(i, nb0 - 1)))
    t1_spec = pl.BlockSpec(
        (k, nb), lambda i: (0, jnp.maximum(i - nb0, 0)))
    edge_spec = pl.BlockSpec((k, nb), lambda i: (0, i))
    w_spec = pl.BlockSpec((k, k), lambda i: (0, 0))
    bias_spec = pl.BlockSpec((k, 1), lambda i: (0, 0))
    return pl.pallas_call(
        functools.partial(_mlp_body, nb0),
        grid=(grid,),
        in_specs=[t0_spec, t1_spec, edge_spec,
                  w_spec, w_spec, w_spec, bias_spec, bias_spec],
        out_specs=edge_spec,
        out_shape=jax.ShapeDtypeStruct((k, n_edges), jnp.float32),
    )(T0, T1, ET, QT, W1E, W2, c1c, b2c)


def kernel(V, E, edge_index, U, Vm, P, Pb, W1, b1, W2, b2):
    n_edges, k = E.shape
    half = n_edges // 2
    pack = 128 // k
    A, B = _precompute(V, U.T, Vm.T)

    t16s = []
    for h in range(2):
        Asrc, Bdst = _sc_gather(A, B, edge_index, h * half, half)
        t_p = _tanh_packed(Asrc.reshape(half // pack, 128),
                           Bdst.reshape(half // pack, 128))
        t16s.append(t_p.reshape(half, k).T)

    ET = E.T
    W1h_T = W1[:, :k].T
    Q = P.T @ W1h_T
    c1 = Pb @ W1h_T + b1
    out_t = _mlp_t(t16s[0], t16s[1], ET, Q.T, W1[:, k:], W2,
                   c1.reshape(k, 1), b2.reshape(k, 1))
    return out_t.T
